# baseline (speedup 1.0000x reference)
import functools

import jax
import jax.numpy as jnp
from jax.experimental import pallas as pl
from jax.experimental.pallas import tpu as pltpu

E = 8
TOP_K = 2
N_GROUP = 4
TOPK_GROUP = 2
D = 1024
FF = 512
FFS = 2048
SCALE = 2.5
T = 2048

BT = 256
BTS = 512


def _router_kernel(h_ref, rw_ref, cw_ref):
    h = h_ref[...]
    rw = rw_ref[...]
    logits = jax.lax.dot_general(
        h, rw, (((1,), (1,)), ((), ())),
        preferred_element_type=jnp.float32,
    )
    s = jax.nn.sigmoid(logits)
    r8 = jax.lax.broadcasted_iota(jnp.int32, (E, E), 0)
    c8 = jax.lax.broadcasted_iota(jnp.int32, (E, E), 1)
    perm = ((r8 ^ 1) == c8).astype(jnp.float32)
    sp = jax.lax.dot_general(
        s, perm, (((1,), (0,)), ((), ())),
        preferred_element_type=jnp.float32,
        precision=jax.lax.Precision.HIGHEST,
    )
    gs = s + sp
    eidx = jax.lax.broadcasted_iota(jnp.int32, (T, E), 1)
    gidx = eidx // 2
    m1 = jnp.max(gs, axis=1, keepdims=True)
    g1 = jnp.min(jnp.where(gs == m1, gidx, N_GROUP), axis=1, keepdims=True)
    gs2 = jnp.where(gidx == g1, -jnp.inf, gs)
    m2 = jnp.max(gs2, axis=1, keepdims=True)
    g2 = jnp.min(jnp.where(gs2 == m2, gidx, N_GROUP), axis=1, keepdims=True)
    sel = (gidx == g1) | (gidx == g2)
    masked = jnp.where(sel, s, 0.0)
    w1 = jnp.max(masked, axis=1, keepdims=True)
    e1 = jnp.min(jnp.where(masked == w1, eidx, E), axis=1, keepdims=True)
    masked2 = jnp.where(eidx == e1, -1.0, masked)
    w2 = jnp.max(masked2, axis=1, keepdims=True)
    e2 = jnp.min(jnp.where(masked2 == w2, eidx, E), axis=1, keepdims=True)
    denom = w1 + w2 + 1e-20
    cw = jnp.where(eidx == e1, w1, 0.0) + jnp.where(eidx == e2, w2, 0.0)
    cw_ref[...] = cw * (SCALE / denom)


def _experts_kernel(h_ref, cw_ref, up_ref, down_ref, out_ref):
    e = pl.program_id(0)
    t = pl.program_id(1)
    rows = pl.ds(t * BT, BT)
    h = h_ref[rows, :]
    up = up_ref[0]
    dn = down_ref[0]
    a = jax.lax.dot_general(h, up, (((1,), (1,)), ((), ())),
                            preferred_element_type=jnp.float32)
    a = a * jax.nn.sigmoid(a)
    y = jax.lax.dot_general(a.astype(jnp.bfloat16), dn, (((1,), (1,)), ((), ())),
                            preferred_element_type=jnp.float32)
    cwb = cw_ref[rows, :]
    eidx = jax.lax.broadcasted_iota(jnp.int32, (BT, E), 1)
    w = jnp.sum(jnp.where(eidx == e, cwb, 0.0), axis=1, keepdims=True)
    contrib = y * w

    @pl.when(e == 0)
    def _():
        out_ref[rows, :] = contrib

    @pl.when(e > 0)
    def _():
        out_ref[rows, :] = out_ref[rows, :] + contrib


def _shared_kernel(routed_ref, h_ref, sup_ref, sdn_ref, out_ref):
    h = h_ref[...]
    a = jax.lax.dot_general(h, sup_ref[...], (((1,), (1,)), ((), ())),
                            preferred_element_type=jnp.float32)
    a = a * jax.nn.sigmoid(a)
    y = jax.lax.dot_general(a.astype(jnp.bfloat16), sdn_ref[...], (((1,), (1,)), ((), ())),
                            preferred_element_type=jnp.float32)
    out_ref[...] = routed_ref[...] + y


def kernel(hidden_states, router_w, up_w, down_w, shared_up_w, shared_down_w):
    orig_shape = hidden_states.shape
    h = hidden_states.reshape(T, D)
    h_bf = h.astype(jnp.bfloat16)
    up_bf = up_w.astype(jnp.bfloat16)
    down_bf = down_w.astype(jnp.bfloat16)
    sup_bf = shared_up_w.astype(jnp.bfloat16)
    sdn_bf = shared_down_w.astype(jnp.bfloat16)

    cw = pl.pallas_call(
        _router_kernel,
        out_shape=jax.ShapeDtypeStruct((T, E), jnp.float32),
        in_specs=[
            pl.BlockSpec((T, D), lambda: (0, 0)),
            pl.BlockSpec((E, D), lambda: (0, 0)),
        ],
        out_specs=pl.BlockSpec((T, E), lambda: (0, 0)),
        interpret=False,
    )(h, router_w)

    routed = pl.pallas_call(
        _experts_kernel,
        grid=(E, T // BT),
        out_shape=jax.ShapeDtypeStruct((T, D), jnp.float32),
        in_specs=[
            pl.BlockSpec((T, D), lambda e, t: (0, 0)),
            pl.BlockSpec((T, E), lambda e, t: (0, 0)),
            pl.BlockSpec((1, FF, D), lambda e, t: (e, 0, 0)),
            pl.BlockSpec((1, D, FF), lambda e, t: (e, 0, 0)),
        ],
        out_specs=pl.BlockSpec((T, D), lambda e, t: (0, 0)),
        compiler_params=pltpu.CompilerParams(
            dimension_semantics=("arbitrary", "arbitrary"),
        ),
        interpret=False,
    )(h_bf, cw, up_bf, down_bf)

    out = pl.pallas_call(
        _shared_kernel,
        grid=(T // BTS,),
        out_shape=jax.ShapeDtypeStruct((T, D), jnp.float32),
        in_specs=[
            pl.BlockSpec((BTS, D), lambda t: (t, 0)),
            pl.BlockSpec((BTS, D), lambda t: (t, 0)),
            pl.BlockSpec((FFS, D), lambda t: (0, 0)),
            pl.BlockSpec((D, FFS), lambda t: (0, 0)),
        ],
        out_specs=pl.BlockSpec((BTS, D), lambda t: (t, 0)),
        compiler_params=pltpu.CompilerParams(
            dimension_semantics=("arbitrary",),
        ),
        interpret=False,
    )(routed, h_bf, sup_bf, sdn_bf)

    return out.reshape(orig_shape)


# speedup vs baseline: 1.2322x; 1.2322x over previous
import functools

import jax
import jax.numpy as jnp
from jax.experimental import pallas as pl
from jax.experimental.pallas import tpu as pltpu

E = 8
TOP_K = 2
N_GROUP = 4
TOPK_GROUP = 2
D = 1024
FF = 512
FFS = 2048
SCALE = 2.5
T = 2048

BT = 256
BTS = 512


def _router_kernel(h_ref, rw_ref, cw_ref):
    h = h_ref[...]
    rw = rw_ref[...]
    logits = jax.lax.dot_general(
        h, rw, (((1,), (1,)), ((), ())),
        preferred_element_type=jnp.float32,
    )
    s = jax.nn.sigmoid(logits)
    r8 = jax.lax.broadcasted_iota(jnp.int32, (E, E), 0)
    c8 = jax.lax.broadcasted_iota(jnp.int32, (E, E), 1)
    perm = ((r8 ^ 1) == c8).astype(jnp.float32)
    sp = jax.lax.dot_general(
        s, perm, (((1,), (0,)), ((), ())),
        preferred_element_type=jnp.float32,
        precision=jax.lax.Precision.HIGHEST,
    )
    gs = s + sp
    eidx = jax.lax.broadcasted_iota(jnp.int32, (T, E), 1)
    gidx = eidx // 2
    m1 = jnp.max(gs, axis=1, keepdims=True)
    g1 = jnp.min(jnp.where(gs == m1, gidx, N_GROUP), axis=1, keepdims=True)
    gs2 = jnp.where(gidx == g1, -jnp.inf, gs)
    m2 = jnp.max(gs2, axis=1, keepdims=True)
    g2 = jnp.min(jnp.where(gs2 == m2, gidx, N_GROUP), axis=1, keepdims=True)
    sel = (gidx == g1) | (gidx == g2)
    masked = jnp.where(sel, s, 0.0)
    w1 = jnp.max(masked, axis=1, keepdims=True)
    e1 = jnp.min(jnp.where(masked == w1, eidx, E), axis=1, keepdims=True)
    masked2 = jnp.where(eidx == e1, -1.0, masked)
    w2 = jnp.max(masked2, axis=1, keepdims=True)
    e2 = jnp.min(jnp.where(masked2 == w2, eidx, E), axis=1, keepdims=True)
    denom = w1 + w2 + 1e-20
    cw = jnp.where(eidx == e1, w1, 0.0) + jnp.where(eidx == e2, w2, 0.0)
    cw_ref[...] = cw * (SCALE / denom)


def _experts_kernel(h_ref, cw_ref, up_ref, down_ref, out_ref):
    e = pl.program_id(0)
    t = pl.program_id(1)
    rows = pl.ds(t * BT, BT)
    h = h_ref[rows, :]
    up = up_ref[0]
    dn = down_ref[0]
    a = jax.lax.dot_general(h, up, (((1,), (1,)), ((), ())),
                            preferred_element_type=jnp.float32)
    a = a * jax.nn.sigmoid(a)
    y = jax.lax.dot_general(a, dn, (((1,), (1,)), ((), ())),
                            preferred_element_type=jnp.float32)
    cwb = cw_ref[rows, :]
    eidx = jax.lax.broadcasted_iota(jnp.int32, (BT, E), 1)
    w = jnp.sum(jnp.where(eidx == e, cwb, 0.0), axis=1, keepdims=True)
    contrib = y * w

    @pl.when(e == 0)
    def _():
        out_ref[rows, :] = contrib

    @pl.when(e > 0)
    def _():
        out_ref[rows, :] = out_ref[rows, :] + contrib


def _shared_kernel(routed_ref, h_ref, sup_ref, sdn_ref, out_ref):
    h = h_ref[...]
    a = jax.lax.dot_general(h, sup_ref[...], (((1,), (1,)), ((), ())),
                            preferred_element_type=jnp.float32)
    a = a * jax.nn.sigmoid(a)
    y = jax.lax.dot_general(a, sdn_ref[...], (((1,), (1,)), ((), ())),
                            preferred_element_type=jnp.float32)
    out_ref[...] = routed_ref[...] + y


def kernel(hidden_states, router_w, up_w, down_w, shared_up_w, shared_down_w):
    orig_shape = hidden_states.shape
    h = hidden_states.reshape(T, D)

    cw = pl.pallas_call(
        _router_kernel,
        out_shape=jax.ShapeDtypeStruct((T, E), jnp.float32),
        in_specs=[
            pl.BlockSpec((T, D), lambda: (0, 0)),
            pl.BlockSpec((E, D), lambda: (0, 0)),
        ],
        out_specs=pl.BlockSpec((T, E), lambda: (0, 0)),
        interpret=False,
    )(h, router_w)

    routed = pl.pallas_call(
        _experts_kernel,
        grid=(E, T // BT),
        out_shape=jax.ShapeDtypeStruct((T, D), jnp.float32),
        in_specs=[
            pl.BlockSpec((T, D), lambda e, t: (0, 0)),
            pl.BlockSpec((T, E), lambda e, t: (0, 0)),
            pl.BlockSpec((1, FF, D), lambda e, t: (e, 0, 0)),
            pl.BlockSpec((1, D, FF), lambda e, t: (e, 0, 0)),
        ],
        out_specs=pl.BlockSpec((T, D), lambda e, t: (0, 0)),
        compiler_params=pltpu.CompilerParams(
            dimension_semantics=("arbitrary", "arbitrary"),
        ),
        interpret=False,
    )(h, cw, up_w, down_w)

    out = pl.pallas_call(
        _shared_kernel,
        grid=(T // BTS,),
        out_shape=jax.ShapeDtypeStruct((T, D), jnp.float32),
        in_specs=[
            pl.BlockSpec((BTS, D), lambda t: (t, 0)),
            pl.BlockSpec((BTS, D), lambda t: (t, 0)),
            pl.BlockSpec((FFS, D), lambda t: (0, 0)),
            pl.BlockSpec((D, FFS), lambda t: (0, 0)),
        ],
        out_specs=pl.BlockSpec((BTS, D), lambda t: (t, 0)),
        compiler_params=pltpu.CompilerParams(
            dimension_semantics=("arbitrary",),
        ),
        interpret=False,
    )(routed, h, shared_up_w, shared_down_w)

    return out.reshape(orig_shape)
